# baseline (speedup 1.0000x reference)
import functools

import jax
import jax.numpy as jnp
from jax.experimental import pallas as pl
from jax.experimental.pallas import tpu as pltpu

D = 32
G4 = 4 * D
N_MLP = 2
N_ROUNDS = 4
FB = 1.0
EPS = 1e-5
NB = 8


def _relu_mlp(x, Ws, bs):
    for l in range(N_MLP):
        x = jnp.dot(x, Ws[l], preferred_element_type=jnp.float32)
        x = jnp.maximum(x + bs[l:l + 1, :], 0.0)
    return x


def _gated_update(pre, c, gamma, beta, gc, bc, lnmat, fbias, dmat):
    mu = jnp.dot(pre, lnmat, preferred_element_type=jnp.float32)
    d = pre - mu
    var = jnp.dot(d * d, lnmat, preferred_element_type=jnp.float32)
    z = d * jax.lax.rsqrt(var + EPS) * gamma + beta
    sg = jax.nn.sigmoid(z + fbias)
    g = jnp.tanh(z[:, 2 * D:3 * D])
    c_new = sg[:, D:2 * D] * c + sg[:, 0:D] * g
    mu2 = jnp.dot(c_new, dmat, preferred_element_type=jnp.float32)
    d2 = c_new - mu2
    v2 = jnp.dot(d2 * d2, dmat, preferred_element_type=jnp.float32)
    h = jnp.tanh(d2 * jax.lax.rsqrt(v2 + EPS) * gc + bc) * sg[:, 3 * D:4 * D]
    return h, c_new


def _msgpass_kernel(cnt_ref, adj_ref,
                    lpos_ref, lneg_ref, cinit_ref,
                    lcW_ref, lcb_ref, clW_ref, clb_ref,
                    cwih_ref, cwhh_ref, cb_ref, cg_ref, cbe_ref, cgc_ref, cbc_ref,
                    lwih_ref, lwhh_ref, lb_ref, lg_ref, lbe_ref, lgc_ref, lbc_ref,
                    rankw_ref, out_ref, *, nb):
    g0 = pl.program_id(0) * nb
    _, L, C = adj_ref.shape
    halfL = L // 2

    ns = [cnt_ref[g0 + i] for i in range(nb)]
    halves = [jax.lax.div(n, jnp.int32(2)) for n in ns]
    adjs = [adj_ref[i] for i in range(nb)]

    r128 = jax.lax.broadcasted_iota(jnp.int32, (G4, G4), 0)
    c128 = jax.lax.broadcasted_iota(jnp.int32, (G4, G4), 1)
    lnmat = jnp.where((r128 // D) == (c128 // D), 1.0 / D, 0.0)
    dmat = jnp.full((D, D), 1.0 / D, jnp.float32)
    lane = jax.lax.broadcasted_iota(jnp.int32, (1, G4), 1)
    fbias = jnp.where((lane >= D) & (lane < 2 * D), FB, 0.0)

    rowL = jax.lax.broadcasted_iota(jnp.int32, (nb * L, D), 0)
    L_h = jnp.where((rowL % L) < halfL,
                    jnp.broadcast_to(lpos_ref[...], (nb * L, D)),
                    jnp.broadcast_to(lneg_ref[...], (nb * L, D)))
    C_h = jnp.broadcast_to(cinit_ref[...], (nb * C, D))
    L_c = jnp.zeros((nb * L, D), jnp.float32)
    C_c = jnp.zeros((nb * C, D), jnp.float32)

    lcW = lcW_ref[...]; lcb = lcb_ref[...]
    clW = clW_ref[...]; clb = clb_ref[...]
    cwih = cwih_ref[...]; cwhh = cwhh_ref[...]; cbias = cb_ref[...]
    cgam = cg_ref[...]; cbet = cbe_ref[...]; cgc = cgc_ref[...]; cbc = cbc_ref[...]
    lwih = lwih_ref[...]; lwhh = lwhh_ref[...]; lbias = lb_ref[...]
    lgam = lg_ref[...]; lbet = lbe_ref[...]; lgc = lgc_ref[...]; lbc = lbc_ref[...]
    lwih_msg = lwih[:D]
    lwih_flip = lwih[D:2 * D]

    rr = jax.lax.broadcasted_iota(jnp.int32, (L, D), 0)

    for _ in range(N_ROUNDS):
        mL = _relu_mlp(L_h, lcW, lcb).astype(jnp.bfloat16)
        lc = jnp.concatenate(
            [jax.lax.dot_general(adjs[i], mL[i * L:(i + 1) * L],
                                 (((0,), (0,)), ((), ())),
                                 preferred_element_type=jnp.float32)
             for i in range(nb)], axis=0)
        pre_c = (jnp.dot(lc, cwih, preferred_element_type=jnp.float32)
                 + jnp.dot(C_h, cwhh, preferred_element_type=jnp.float32)
                 + cbias)
        C_h, C_c = _gated_update(pre_c, C_c, cgam, cbet, cgc, cbc,
                                 lnmat, fbias, dmat)

        mC = _relu_mlp(C_h, clW, clb).astype(jnp.bfloat16)
        cl = jnp.concatenate(
            [jnp.dot(adjs[i], mC[i * C:(i + 1) * C],
                     preferred_element_type=jnp.float32)
             for i in range(nb)], axis=0)

        flips = []
        for i in range(nb):
            lh_i = L_h[i * L:(i + 1) * L]
            dn = pltpu.roll(lh_i, -halves[i], axis=0)
            up = pltpu.roll(lh_i, halves[i], axis=0)
            flips.append(jnp.where(rr < halves[i], dn,
                                   jnp.where(rr < ns[i], up, 0.0)))
        flipped = jnp.concatenate(flips, axis=0)

        pre_l = (jnp.dot(cl, lwih_msg, preferred_element_type=jnp.float32)
                 + jnp.dot(flipped, lwih_flip, preferred_element_type=jnp.float32)
                 + jnp.dot(L_h, lwhh, preferred_element_type=jnp.float32)
                 + lbias)
        L_h, L_c = _gated_update(pre_l, L_c, lgam, lbet, lgc, lbc,
                                 lnmat, fbias, dmat)

    w = rankw_ref[...]
    rh = jax.lax.broadcasted_iota(jnp.int32, (halfL, 1), 0)
    col2 = jax.lax.broadcasted_iota(jnp.int32, (halfL, 2), 1)
    for i in range(nb):
        top = L_h[i * L:i * L + halfL]
        s = jnp.sum(top * w, axis=-1, keepdims=True)
        m = (rh < halves[i]).astype(jnp.float32)
        out_ref[i] = jnp.where(col2 == 0, jnp.tanh(0.5 * s) * m, s * m)


def kernel(adjacency, batch_lit_counts, L_pos_init, L_neg_init, C_init,
           lc_W, lc_b, cl_W, cl_b,
           C_wih, C_whh, C_bias, C_gamma, C_beta, C_gc, C_bc,
           L_wih, L_whh, L_bias, L_gamma, L_beta, L_gc, L_bc, rank_w):
    B, L, C = adjacency.shape
    nb = NB
    while B % nb:
        nb //= 2
    counts = jnp.asarray(batch_lit_counts, jnp.int32)
    adjacency = adjacency.astype(jnp.bfloat16)

    args = (adjacency, L_pos_init, L_neg_init, C_init,
            lc_W, lc_b, cl_W, cl_b,
            C_wih, C_whh, C_bias, C_gamma, C_beta, C_gc, C_bc,
            L_wih, L_whh, L_bias, L_gamma, L_beta, L_gc, L_bc, rank_w)

    def whole(a):
        nd = a.ndim
        return pl.BlockSpec(a.shape, lambda b, cnt, _nd=nd: (0,) * _nd)

    in_specs = ([pl.BlockSpec((nb, L, C), lambda b, cnt: (b, 0, 0))]
                + [whole(a) for a in args[1:]])

    out = pl.pallas_call(
        functools.partial(_msgpass_kernel, nb=nb),
        out_shape=jax.ShapeDtypeStruct((B, L // 2, 2), jnp.float32),
        grid_spec=pltpu.PrefetchScalarGridSpec(
            num_scalar_prefetch=1,
            grid=(B // nb,),
            in_specs=in_specs,
            out_specs=pl.BlockSpec((nb, L // 2, 2), lambda b, cnt: (b, 0, 0)),
        ),
        compiler_params=pltpu.CompilerParams(dimension_semantics=("parallel",)),
    )(counts, *args)

    return out[:, :, 0:1], out[:, :, 1:2]


# speedup vs baseline: 1.0966x; 1.0966x over previous
import functools

import jax
import jax.numpy as jnp
from jax.experimental import pallas as pl
from jax.experimental.pallas import tpu as pltpu

D = 32
G4 = 4 * D
N_MLP = 2
N_ROUNDS = 4
FB = 1.0
EPS = 1e-5
NB = 16


def _relu_mlp(x, Ws, bs):
    for l in range(N_MLP):
        x = jnp.dot(x, Ws[l], preferred_element_type=jnp.float32)
        x = jnp.maximum(x + bs[l:l + 1, :], 0.0)
    return x


def _gated_update(pre, c, gamma, beta, gc, bc, lnmat, fbias, dmat):
    mu = jnp.dot(pre, lnmat, preferred_element_type=jnp.float32)
    d = pre - mu
    var = jnp.dot(d * d, lnmat, preferred_element_type=jnp.float32)
    z = d * jax.lax.rsqrt(var + EPS) * gamma + beta
    sg = jax.nn.sigmoid(z + fbias)
    g = jnp.tanh(z[:, 2 * D:3 * D])
    c_new = sg[:, D:2 * D] * c + sg[:, 0:D] * g
    mu2 = jnp.dot(c_new, dmat, preferred_element_type=jnp.float32)
    d2 = c_new - mu2
    v2 = jnp.dot(d2 * d2, dmat, preferred_element_type=jnp.float32)
    h = jnp.tanh(d2 * jax.lax.rsqrt(v2 + EPS) * gc + bc) * sg[:, 3 * D:4 * D]
    return h, c_new


def _msgpass_kernel(cnt_ref, adj_ref,
                    lpos_ref, lneg_ref, cinit_ref,
                    lcW_ref, lcb_ref, clW_ref, clb_ref,
                    cwih_ref, cwhh_ref, cb_ref, cg_ref, cbe_ref, cgc_ref, cbc_ref,
                    lwih_ref, lwhh_ref, lb_ref, lg_ref, lbe_ref, lgc_ref, lbc_ref,
                    rankw_ref, out_ref, *, nb):
    g0 = pl.program_id(0) * nb
    _, L, C = adj_ref.shape
    halfL = L // 2

    ns = [cnt_ref[g0 + i] for i in range(nb)]
    halves = [jax.lax.div(n, jnp.int32(2)) for n in ns]
    adjs = [adj_ref[i] for i in range(nb)]

    r128 = jax.lax.broadcasted_iota(jnp.int32, (G4, G4), 0)
    c128 = jax.lax.broadcasted_iota(jnp.int32, (G4, G4), 1)
    lnmat = jnp.where((r128 // D) == (c128 // D), 1.0 / D, 0.0)
    dmat = jnp.full((D, D), 1.0 / D, jnp.float32)
    lane = jax.lax.broadcasted_iota(jnp.int32, (1, G4), 1)
    fbias = jnp.where((lane >= D) & (lane < 2 * D), FB, 0.0)

    rowL = jax.lax.broadcasted_iota(jnp.int32, (nb * L, D), 0)
    L_h = jnp.where((rowL % L) < halfL,
                    jnp.broadcast_to(lpos_ref[...], (nb * L, D)),
                    jnp.broadcast_to(lneg_ref[...], (nb * L, D)))
    C_h = jnp.broadcast_to(cinit_ref[...], (nb * C, D))
    L_c = jnp.zeros((nb * L, D), jnp.float32)
    C_c = jnp.zeros((nb * C, D), jnp.float32)

    lcW = lcW_ref[...]; lcb = lcb_ref[...]
    clW = clW_ref[...]; clb = clb_ref[...]
    cwih = cwih_ref[...]; cwhh = cwhh_ref[...]; cbias = cb_ref[...]
    cgam = cg_ref[...]; cbet = cbe_ref[...]; cgc = cgc_ref[...]; cbc = cbc_ref[...]
    lwih = lwih_ref[...]; lwhh = lwhh_ref[...]; lbias = lb_ref[...]
    lgam = lg_ref[...]; lbet = lbe_ref[...]; lgc = lgc_ref[...]; lbc = lbc_ref[...]
    lwih_msg = lwih[:D]
    lwih_flip = lwih[D:2 * D]

    rr = jax.lax.broadcasted_iota(jnp.int32, (L, D), 0)

    for _ in range(N_ROUNDS):
        mL = _relu_mlp(L_h, lcW, lcb)
        lc = jnp.concatenate(
            [jax.lax.dot_general(adjs[i], mL[i * L:(i + 1) * L],
                                 (((0,), (0,)), ((), ())),
                                 preferred_element_type=jnp.float32)
             for i in range(nb)], axis=0)
        pre_c = (jnp.dot(lc, cwih, preferred_element_type=jnp.float32)
                 + jnp.dot(C_h, cwhh, preferred_element_type=jnp.float32)
                 + cbias)
        C_h, C_c = _gated_update(pre_c, C_c, cgam, cbet, cgc, cbc,
                                 lnmat, fbias, dmat)

        mC = _relu_mlp(C_h, clW, clb)
        cl = jnp.concatenate(
            [jnp.dot(adjs[i], mC[i * C:(i + 1) * C],
                     preferred_element_type=jnp.float32)
             for i in range(nb)], axis=0)

        flips = []
        for i in range(nb):
            lh_i = L_h[i * L:(i + 1) * L]
            dn = pltpu.roll(lh_i, -halves[i], axis=0)
            up = pltpu.roll(lh_i, halves[i], axis=0)
            flips.append(jnp.where(rr < halves[i], dn,
                                   jnp.where(rr < ns[i], up, 0.0)))
        flipped = jnp.concatenate(flips, axis=0)

        pre_l = (jnp.dot(cl, lwih_msg, preferred_element_type=jnp.float32)
                 + jnp.dot(flipped, lwih_flip, preferred_element_type=jnp.float32)
                 + jnp.dot(L_h, lwhh, preferred_element_type=jnp.float32)
                 + lbias)
        L_h, L_c = _gated_update(pre_l, L_c, lgam, lbet, lgc, lbc,
                                 lnmat, fbias, dmat)

    w = rankw_ref[...]
    rh = jax.lax.broadcasted_iota(jnp.int32, (halfL, 1), 0)
    col2 = jax.lax.broadcasted_iota(jnp.int32, (halfL, 2), 1)
    for i in range(nb):
        top = L_h[i * L:i * L + halfL]
        s = jnp.sum(top * w, axis=-1, keepdims=True)
        m = (rh < halves[i]).astype(jnp.float32)
        out_ref[i] = jnp.where(col2 == 0, jnp.tanh(0.5 * s) * m, s * m)


def kernel(adjacency, batch_lit_counts, L_pos_init, L_neg_init, C_init,
           lc_W, lc_b, cl_W, cl_b,
           C_wih, C_whh, C_bias, C_gamma, C_beta, C_gc, C_bc,
           L_wih, L_whh, L_bias, L_gamma, L_beta, L_gc, L_bc, rank_w):
    B, L, C = adjacency.shape
    nb = NB
    while B % nb:
        nb //= 2
    counts = jnp.asarray(batch_lit_counts, jnp.int32)

    args = (adjacency, L_pos_init, L_neg_init, C_init,
            lc_W, lc_b, cl_W, cl_b,
            C_wih, C_whh, C_bias, C_gamma, C_beta, C_gc, C_bc,
            L_wih, L_whh, L_bias, L_gamma, L_beta, L_gc, L_bc, rank_w)

    def whole(a):
        nd = a.ndim
        return pl.BlockSpec(a.shape, lambda b, cnt, _nd=nd: (0,) * _nd)

    in_specs = ([pl.BlockSpec((nb, L, C), lambda b, cnt: (b, 0, 0))]
                + [whole(a) for a in args[1:]])

    out = pl.pallas_call(
        functools.partial(_msgpass_kernel, nb=nb),
        out_shape=jax.ShapeDtypeStruct((B, L // 2, 2), jnp.float32),
        grid_spec=pltpu.PrefetchScalarGridSpec(
            num_scalar_prefetch=1,
            grid=(B // nb,),
            in_specs=in_specs,
            out_specs=pl.BlockSpec((nb, L // 2, 2), lambda b, cnt: (b, 0, 0)),
        ),
        compiler_params=pltpu.CompilerParams(dimension_semantics=("parallel",)),
    )(counts, *args)

    return out[:, :, 0:1], out[:, :, 1:2]


# speedup vs baseline: 2.6953x; 2.4578x over previous
import functools

import jax
import jax.numpy as jnp
from jax.experimental import pallas as pl
from jax.experimental.pallas import tpu as pltpu

D = 32
G4 = 4 * D
N_MLP = 2
N_ROUNDS = 4
FB = 1.0
EPS = 1e-5
NB = 16


def _relu_mlp(x, Ws, bs):
    for l in range(N_MLP):
        x = jnp.dot(Ws[l], x, preferred_element_type=jnp.float32)
        x = jnp.maximum(x + bs[l], 0.0)
    return x


def _gated_update(pre, c, gamma, beta, gc, bc, lnmat, fbias, dmat):
    mu = jnp.dot(lnmat, pre, preferred_element_type=jnp.float32)
    d = pre - mu
    var = jnp.dot(lnmat, d * d, preferred_element_type=jnp.float32)
    z = d * jax.lax.rsqrt(var + EPS) * gamma + beta
    sg = jax.nn.sigmoid(z + fbias)
    g = jnp.tanh(z[2 * D:3 * D])
    c_new = sg[D:2 * D] * c + sg[0:D] * g
    mu2 = jnp.dot(dmat, c_new, preferred_element_type=jnp.float32)
    d2 = c_new - mu2
    v2 = jnp.dot(dmat, d2 * d2, preferred_element_type=jnp.float32)
    h = jnp.tanh(d2 * jax.lax.rsqrt(v2 + EPS) * gc + bc) * sg[3 * D:4 * D]
    return h, c_new


def _msgpass_kernel(cnt_ref, adj_ref,
                    lpos_ref, lneg_ref, cinit_ref,
                    lcW_ref, lcb_ref, clW_ref, clb_ref,
                    cwih_ref, cwhh_ref, cb_ref, cg_ref, cbe_ref, cgc_ref, cbc_ref,
                    lwihm_ref, lwihf_ref, lwhh_ref, lb_ref, lg_ref, lbe_ref, lgc_ref, lbc_ref,
                    rankw_ref, out_ref, *, nb):
    g0 = pl.program_id(0) * nb
    _, L, C = adj_ref.shape
    halfL = L // 2

    ns = [cnt_ref[g0 + i] for i in range(nb)]
    halves = [jax.lax.div(n, jnp.int32(2)) for n in ns]
    adjs = [adj_ref[i] for i in range(nb)]

    r128 = jax.lax.broadcasted_iota(jnp.int32, (G4, G4), 0)
    c128 = jax.lax.broadcasted_iota(jnp.int32, (G4, G4), 1)
    lnmat = jnp.where((r128 // D) == (c128 // D), 1.0 / D, 0.0)
    dmat = jnp.full((D, D), 1.0 / D, jnp.float32)
    srow = jax.lax.broadcasted_iota(jnp.int32, (G4, 1), 0)
    fbias = jnp.where((srow >= D) & (srow < 2 * D), FB, 0.0)

    colL = jax.lax.broadcasted_iota(jnp.int32, (D, nb * L), 1)
    L_h = jnp.where((colL % L) < halfL,
                    jnp.broadcast_to(lpos_ref[...], (D, nb * L)),
                    jnp.broadcast_to(lneg_ref[...], (D, nb * L)))
    C_h = jnp.broadcast_to(cinit_ref[...], (D, nb * C))
    L_c = jnp.zeros((D, nb * L), jnp.float32)
    C_c = jnp.zeros((D, nb * C), jnp.float32)

    lcW = lcW_ref[...]; lcb = lcb_ref[...]
    clW = clW_ref[...]; clb = clb_ref[...]
    cwih = cwih_ref[...]; cwhh = cwhh_ref[...]; cbias = cb_ref[...]
    cgam = cg_ref[...]; cbet = cbe_ref[...]; cgc = cgc_ref[...]; cbc = cbc_ref[...]
    lwihm = lwihm_ref[...]; lwihf = lwihf_ref[...]; lwhh = lwhh_ref[...]
    lbias = lb_ref[...]
    lgam = lg_ref[...]; lbet = lbe_ref[...]; lgc = lgc_ref[...]; lbc = lbc_ref[...]

    cc = jax.lax.broadcasted_iota(jnp.int32, (D, L), 1)

    for _ in range(N_ROUNDS):
        mL = _relu_mlp(L_h, lcW, lcb)
        lc = jnp.concatenate(
            [jnp.dot(mL[:, i * L:(i + 1) * L], adjs[i],
                     preferred_element_type=jnp.float32)
             for i in range(nb)], axis=1)
        pre_c = (jnp.dot(cwih, lc, preferred_element_type=jnp.float32)
                 + jnp.dot(cwhh, C_h, preferred_element_type=jnp.float32)
                 + cbias)
        C_h, C_c = _gated_update(pre_c, C_c, cgam, cbet, cgc, cbc,
                                 lnmat, fbias, dmat)

        mC = _relu_mlp(C_h, clW, clb)
        cl = jnp.concatenate(
            [jax.lax.dot_general(mC[:, i * C:(i + 1) * C], adjs[i],
                                 (((1,), (1,)), ((), ())),
                                 preferred_element_type=jnp.float32)
             for i in range(nb)], axis=1)

        flips = []
        for i in range(nb):
            lh_i = L_h[:, i * L:(i + 1) * L]
            dn = pltpu.roll(lh_i, -halves[i], axis=1)
            up = pltpu.roll(lh_i, halves[i], axis=1)
            flips.append(jnp.where(cc < halves[i], dn,
                                   jnp.where(cc < ns[i], up, 0.0)))
        flipped = jnp.concatenate(flips, axis=1)

        pre_l = (jnp.dot(lwihm, cl, preferred_element_type=jnp.float32)
                 + jnp.dot(lwihf, flipped, preferred_element_type=jnp.float32)
                 + jnp.dot(lwhh, L_h, preferred_element_type=jnp.float32)
                 + lbias)
        L_h, L_c = _gated_update(pre_l, L_c, lgam, lbet, lgc, lbc,
                                 lnmat, fbias, dmat)

    w = rankw_ref[...]
    rh = jax.lax.broadcasted_iota(jnp.int32, (halfL, 1), 0)
    col2 = jax.lax.broadcasted_iota(jnp.int32, (halfL, 2), 1)
    for i in range(nb):
        top = L_h[:, i * L:i * L + halfL]
        s = jax.lax.dot_general(top, w, (((0,), (1,)), ((), ())),
                                preferred_element_type=jnp.float32)
        m = (rh < halves[i]).astype(jnp.float32)
        out_ref[i] = jnp.where(col2 == 0, jnp.tanh(0.5 * s) * m, s * m)


def kernel(adjacency, batch_lit_counts, L_pos_init, L_neg_init, C_init,
           lc_W, lc_b, cl_W, cl_b,
           C_wih, C_whh, C_bias, C_gamma, C_beta, C_gc, C_bc,
           L_wih, L_whh, L_bias, L_gamma, L_beta, L_gc, L_bc, rank_w):
    B, L, C = adjacency.shape
    nb = NB
    while B % nb:
        nb //= 2
    counts = jnp.asarray(batch_lit_counts, jnp.int32)

    tv = lambda v: v.T
    tm = lambda m: m.T
    args = (adjacency,
            tv(L_pos_init), tv(L_neg_init), tv(C_init),
            jnp.transpose(lc_W, (0, 2, 1)), lc_b[:, :, None],
            jnp.transpose(cl_W, (0, 2, 1)), cl_b[:, :, None],
            tm(C_wih), tm(C_whh), tv(C_bias), tv(C_gamma), tv(C_beta),
            tv(C_gc), tv(C_bc),
            tm(L_wih[:D]), tm(L_wih[D:2 * D]), tm(L_whh), tv(L_bias),
            tv(L_gamma), tv(L_beta), tv(L_gc), tv(L_bc),
            rank_w)

    def whole(a):
        nd = a.ndim
        return pl.BlockSpec(a.shape, lambda b, cnt, _nd=nd: (0,) * _nd)

    in_specs = ([pl.BlockSpec((nb, L, C), lambda b, cnt: (b, 0, 0))]
                + [whole(a) for a in args[1:]])

    out = pl.pallas_call(
        functools.partial(_msgpass_kernel, nb=nb),
        out_shape=jax.ShapeDtypeStruct((B, L // 2, 2), jnp.float32),
        grid_spec=pltpu.PrefetchScalarGridSpec(
            num_scalar_prefetch=1,
            grid=(B // nb,),
            in_specs=in_specs,
            out_specs=pl.BlockSpec((nb, L // 2, 2), lambda b, cnt: (b, 0, 0)),
        ),
        compiler_params=pltpu.CompilerParams(dimension_semantics=("parallel",)),
    )(counts, *args)

    return out[:, :, 0:1], out[:, :, 1:2]
